# baseline (speedup 1.0000x reference)
import functools

import jax
import jax.numpy as jnp
from jax import lax
from jax.experimental import pallas as pl
from jax.experimental.pallas import tpu as pltpu
from jax.experimental.pallas import tpu_sc as plsc

BLK = 64
D = 1024
W = 512
L = 16
UPW = 16


def _sc_body(states, maskr, diff, mean, cmask,
             b0, b1, b2, mall, meanall, cmbuf,
             is0, is1, is2, os0, os1, os2):
    bufs = [b0, b1, b2]
    isems = [is0, is1, is2]
    osems = [os0, os1, os2]
    wid = lax.axis_index("s") * 2 + lax.axis_index("c")
    lanes = lax.iota(jnp.int32, L)
    base = wid * UPW

    pltpu.sync_copy(maskr.at[pl.ds(base, UPW)], mall)
    pltpu.async_copy(states.at[base, :, pl.ds(0, W)], b0, is0)
    pltpu.async_copy(states.at[base, :, pl.ds(W, W)], b1, is1)

    def compute(buf, ms, rcp, i, mo):
        def col(c, carry):
            o = pl.multiple_of(c * L, L)
            acc = [jnp.zeros((L,), jnp.float32) for _ in range(4)]
            for r in range(BLK):
                acc[r % 4] = acc[r % 4] + buf[r, pl.ds(o, L)] * ms[r]
            mc = ((acc[0] + acc[1]) + (acc[2] + acc[3])) * rcp
            meanall[i, pl.ds(mo + o, L)] = mc
            for r in range(BLK):
                buf[r, pl.ds(o, L)] = mc - buf[r, pl.ds(o, L)] * ms[r]
            return carry

        lax.fori_loop(0, W // L, col, 0)

    def make_variant(a, bb, c):
        A, B, C = bufs[a], bufs[bb], bufs[c]
        iA, iB, iC = isems[a], isems[bb], isems[c]
        oA, oB, oC = osems[a], osems[bb], osems[c]

        def variant(args):
            (i,) = args
            u = base + i
            mchunks = [mall[i, pl.ds(k * L, L)] for k in range(BLK // L)]
            ms = [mchunks[r // L][r % L] for r in range(BLK)]
            cnt = functools.reduce(lambda x, y: x + y, ms)
            denom = jnp.full((L,), cnt, jnp.float32) + 1e-4
            rcp = 1.0 / denom

            pltpu.make_async_copy(states.at[u, :, pl.ds(0, W)], A, iA).wait()
            compute(A, ms, rcp, i, 0)
            pltpu.async_copy(A, diff.at[u, :, pl.ds(0, W)], oA)

            @pl.when(i + 1 < UPW)
            def _():
                @pl.when(i > 0)
                def _():
                    pltpu.make_async_copy(
                        C, diff.at[u - 1, :, pl.ds(W, W)], oC).wait()
                pltpu.async_copy(states.at[u + 1, :, pl.ds(0, W)], C, iC)

            pltpu.make_async_copy(states.at[u, :, pl.ds(W, W)], B, iB).wait()
            compute(B, ms, rcp, i, W)
            pltpu.async_copy(B, diff.at[u, :, pl.ds(W, W)], oB)

            @pl.when(i + 1 < UPW)
            def _():
                pltpu.make_async_copy(
                    A, diff.at[u, :, pl.ds(0, W)], oA).wait()
                pltpu.async_copy(states.at[u + 1, :, pl.ds(W, W)], A, iA)

            return cnt

        return variant

    variants = [make_variant(0, 1, 2), make_variant(2, 0, 1),
                make_variant(1, 2, 0)]

    def unit(i, cmvec):
        cnt = lax.switch(i % 3, variants, (i,))
        cmval = (cnt > 0.0).astype(jnp.float32)
        return jnp.where(lanes == i, cmval, cmvec)

    cmvec = lax.fori_loop(0, UPW, unit, jnp.zeros((L,), jnp.float32))

    last = base + UPW - 1
    pltpu.make_async_copy(b2, diff.at[last - 1, :, pl.ds(W, W)], os2).wait()
    pltpu.make_async_copy(b0, diff.at[last, :, pl.ds(0, W)], os0).wait()
    pltpu.make_async_copy(b1, diff.at[last, :, pl.ds(W, W)], os1).wait()
    pltpu.sync_copy(meanall, mean.at[pl.ds(base, UPW)])
    cmbuf[...] = cmvec
    pltpu.sync_copy(cmbuf, cmask.at[pl.ds(base, UPW)])


@jax.jit
def _run(states, mask):
    b, f, d = states.shape
    nb = f // BLK
    n = b * nb
    sr = states.reshape(n, BLK, d)
    mr = mask.reshape(n, BLK)
    mesh = plsc.VectorSubcoreMesh(core_axis_name="c", subcore_axis_name="s")
    diff, mean, cmask = pl.kernel(
        _sc_body,
        mesh=mesh,
        out_type=[
            jax.ShapeDtypeStruct((n, BLK, d), states.dtype),
            jax.ShapeDtypeStruct((n, d), states.dtype),
            jax.ShapeDtypeStruct((n,), states.dtype),
        ],
        scratch_types=[
            pltpu.VMEM((BLK, W), jnp.float32),
            pltpu.VMEM((BLK, W), jnp.float32),
            pltpu.VMEM((BLK, W), jnp.float32),
            pltpu.VMEM((UPW, BLK), jnp.float32),
            pltpu.VMEM((UPW, d), jnp.float32),
            pltpu.VMEM((L,), jnp.float32),
        ] + [pltpu.SemaphoreType.DMA] * 6,
    )(sr, mr)
    return diff, mean, cmask


def kernel(fine_token_states, fine_token_mask):
    b, f, d = fine_token_states.shape
    nb = f // BLK
    diff, mean, cmask = _run(fine_token_states, fine_token_mask)
    indice = jnp.broadcast_to(jnp.arange(nb, dtype=jnp.int32)[None, :], (b, nb))
    return (mean.reshape(b, nb, d), cmask.reshape(b, nb),
            diff.reshape(b, nb, BLK, d), indice)


# speedup vs baseline: 1.1414x; 1.1414x over previous
import functools

import jax
import jax.numpy as jnp
from jax import lax
from jax.experimental import pallas as pl
from jax.experimental.pallas import tpu as pltpu
from jax.experimental.pallas import tpu_sc as plsc

BLK = 64
D = 1024
W = 512
L = 16
UPW = 16


def _sc_body(states, maskr, diff, mean, cmask,
             b0, b1, b2, mall, meanbuf, cmbuf,
             is0, is1, is2, os0, os1, os2, msem):
    bufs = [b0, b1, b2]
    isems = [is0, is1, is2]
    osems = [os0, os1, os2]
    wid = lax.axis_index("s") * 2 + lax.axis_index("c")
    lanes = lax.iota(jnp.int32, L)
    base = wid * UPW

    pltpu.sync_copy(maskr.at[pl.ds(base, UPW)], mall)
    pltpu.async_copy(states.at[base, :, pl.ds(0, W)], b0, is0)
    pltpu.async_copy(states.at[base, :, pl.ds(W, W)], b1, is1)

    def compute(buf, ms, rcp, mo):
        def col(c, carry):
            o = pl.multiple_of(c * L, L)
            acc = [jnp.zeros((L,), jnp.float32) for _ in range(4)]
            for r in range(BLK):
                xm = buf[r, pl.ds(o, L)] * ms[r]
                buf[r, pl.ds(o, L)] = xm
                acc[r % 4] = acc[r % 4] + xm
            mc = ((acc[0] + acc[1]) + (acc[2] + acc[3])) * rcp
            meanbuf[pl.ds(mo + o, L)] = mc
            for r in range(BLK):
                buf[r, pl.ds(o, L)] = mc - buf[r, pl.ds(o, L)]
            return carry

        lax.fori_loop(0, W // L, col, 0)

    def make_variant(a, bb, c):
        A, B, C = bufs[a], bufs[bb], bufs[c]
        iA, iB, iC = isems[a], isems[bb], isems[c]
        oA, oB, oC = osems[a], osems[bb], osems[c]

        def variant(args):
            (i,) = args
            u = base + i
            mchunks = [mall[i, pl.ds(k * L, L)] for k in range(BLK // L)]
            ms = [mchunks[r // L][r % L] for r in range(BLK)]
            cnt = functools.reduce(lambda x, y: x + y, ms)
            denom = jnp.full((L,), cnt, jnp.float32) + 1e-4
            rcp = 1.0 / denom

            @pl.when(i > 0)
            def _():
                pltpu.make_async_copy(meanbuf, mean.at[u - 1], msem).wait()

            pltpu.make_async_copy(states.at[u, :, pl.ds(0, W)], A, iA).wait()
            compute(A, ms, rcp, 0)
            pltpu.async_copy(A, diff.at[u, :, pl.ds(0, W)], oA)

            @pl.when(i + 1 < UPW)
            def _():
                @pl.when(i > 0)
                def _():
                    pltpu.make_async_copy(
                        C, diff.at[u - 1, :, pl.ds(W, W)], oC).wait()
                pltpu.async_copy(states.at[u + 1, :, pl.ds(0, W)], C, iC)

            pltpu.make_async_copy(states.at[u, :, pl.ds(W, W)], B, iB).wait()
            compute(B, ms, rcp, W)
            pltpu.async_copy(B, diff.at[u, :, pl.ds(W, W)], oB)

            @pl.when(i + 1 < UPW)
            def _():
                pltpu.make_async_copy(
                    A, diff.at[u, :, pl.ds(0, W)], oA).wait()
                pltpu.async_copy(states.at[u + 1, :, pl.ds(W, W)], A, iA)

            pltpu.async_copy(meanbuf, mean.at[u], msem)
            return cnt

        return variant

    variants = [make_variant(0, 1, 2), make_variant(2, 0, 1),
                make_variant(1, 2, 0)]

    def unit(i, cmvec):
        cnt = lax.switch(i % 3, variants, (i,))
        cmval = (cnt > 0.0).astype(jnp.float32)
        return jnp.where(lanes == i, cmval, cmvec)

    cmvec = lax.fori_loop(0, UPW, unit, jnp.zeros((L,), jnp.float32))

    last = base + UPW - 1
    pltpu.make_async_copy(meanbuf, mean.at[last], msem).wait()
    pltpu.make_async_copy(b2, diff.at[last - 1, :, pl.ds(W, W)], os2).wait()
    pltpu.make_async_copy(b0, diff.at[last, :, pl.ds(0, W)], os0).wait()
    pltpu.make_async_copy(b1, diff.at[last, :, pl.ds(W, W)], os1).wait()
    cmbuf[...] = cmvec
    pltpu.sync_copy(cmbuf, cmask.at[pl.ds(base, UPW)])


@jax.jit
def _run(states, mask):
    b, f, d = states.shape
    nb = f // BLK
    n = b * nb
    sr = states.reshape(n, BLK, d)
    mr = mask.reshape(n, BLK)
    mesh = plsc.VectorSubcoreMesh(core_axis_name="c", subcore_axis_name="s")
    diff, mean, cmask = pl.kernel(
        _sc_body,
        mesh=mesh,
        out_type=[
            jax.ShapeDtypeStruct((n, BLK, d), states.dtype),
            jax.ShapeDtypeStruct((n, d), states.dtype),
            jax.ShapeDtypeStruct((n,), states.dtype),
        ],
        scratch_types=[
            pltpu.VMEM((BLK, W), jnp.float32),
            pltpu.VMEM((BLK, W), jnp.float32),
            pltpu.VMEM((BLK, W), jnp.float32),
            pltpu.VMEM((UPW, BLK), jnp.float32),
            pltpu.VMEM((d,), jnp.float32),
            pltpu.VMEM((L,), jnp.float32),
        ] + [pltpu.SemaphoreType.DMA] * 7,
    )(sr, mr)
    return diff, mean, cmask


def kernel(fine_token_states, fine_token_mask):
    b, f, d = fine_token_states.shape
    nb = f // BLK
    diff, mean, cmask = _run(fine_token_states, fine_token_mask)
    indice = jnp.broadcast_to(jnp.arange(nb, dtype=jnp.int32)[None, :], (b, nb))
    return (mean.reshape(b, nb, d), cmask.reshape(b, nb),
            diff.reshape(b, nb, BLK, d), indice)
